# Initial kernel scaffold; baseline (speedup 1.0000x reference)
import jax
import jax.numpy as jnp
from jax.experimental import pallas as pl


def kernel(inputs, edge_index, item_emb, attr_emb, enc_W1, enc_b1, enc_W2, enc_b2, conv1_W, conv1_b, conv2_W, conv2_b):
    raise NotImplementedError("write your pallas kernel here")


# speedup vs baseline: 9.4805x; 9.4805x over previous
import functools

import jax
import jax.numpy as jnp
from jax import lax
from jax.experimental import pallas as pl
from jax.experimental.pallas import tpu as pltpu
from jax.experimental.pallas import tpu_sc as plsc

N_ITEMS = 8000
N_ATTRS = 2000
N_NODES = N_ITEMS + N_ATTRS
N_EDGES = 160000
IN_DIM = 512
EMBED_DIM = 256
HALF = EMBED_DIM // 2

NC = 2
NS = 16
CHUNK = 128

EPT_DEG = N_EDGES // (NC * NS)
DEG_FULL = EPT_DEG // CHUNK
DEG_TAIL = EPT_DEG - DEG_FULL * CHUNK

EPC = N_EDGES // NS
AGG_FULL = EPC // CHUNK
AGG_TAIL = EPC - AGG_FULL * CHUNK

ROWS_PER_TILE = 624
ROWS_REM = N_NODES - NS * ROWS_PER_TILE
PAIRS = 4096
GPT = 2 * PAIRS // (NC * NS)

_sc_mesh = plsc.VectorSubcoreMesh(core_axis_name="c", subcore_axis_name="s")


def _enc_body(x_ref, w1_ref, b1_ref, w2_ref, b2_ref, o_ref):
    h = jnp.dot(x_ref[...], w1_ref[...], preferred_element_type=jnp.float32)
    h = h + b1_ref[...]
    h = jnp.where(h > 0, h, jnp.exp(h) - 1.0)
    o_ref[...] = (
        jnp.dot(h, w2_ref[...], preferred_element_type=jnp.float32) + b2_ref[...]
    )


def _encoder(x, w1, b1, w2, b2):
    blk = 1000
    return pl.pallas_call(
        _enc_body,
        grid=(N_NODES // blk,),
        in_specs=[
            pl.BlockSpec((blk, IN_DIM), lambda i: (i, 0)),
            pl.BlockSpec((IN_DIM, IN_DIM), lambda i: (0, 0)),
            pl.BlockSpec((1, IN_DIM), lambda i: (0, 0)),
            pl.BlockSpec((IN_DIM, EMBED_DIM), lambda i: (0, 0)),
            pl.BlockSpec((1, EMBED_DIM), lambda i: (0, 0)),
        ],
        out_specs=pl.BlockSpec((blk, EMBED_DIM), lambda i: (i, 0)),
        out_shape=jax.ShapeDtypeStruct((N_NODES, EMBED_DIM), jnp.float32),
    )(x, w1, b1, w2, b2)


def _dinv_of(deg_ref):
    return lax.rsqrt(deg_ref[:, 0] + deg_ref[:, 1] + 1.0)


def _xform_body(z_ref, w_ref, deg_ref, o_ref):
    dinv = _dinv_of(deg_ref)
    h = jnp.dot(z_ref[...], w_ref[...], preferred_element_type=jnp.float32)
    o_ref[0] = h * dinv[:, None]


def _xform(z, w, deg2):
    blk = 1000
    return pl.pallas_call(
        _xform_body,
        grid=(N_NODES // blk, NC),
        in_specs=[
            pl.BlockSpec((blk, EMBED_DIM), lambda i, j: (i, 0)),
            pl.BlockSpec((EMBED_DIM, HALF), lambda i, j: (0, j)),
            pl.BlockSpec((blk, NC), lambda i, j: (i, 0)),
        ],
        out_specs=pl.BlockSpec((1, blk, HALF), lambda i, j: (j, i, 0)),
        out_shape=jax.ShapeDtypeStruct((NC, N_NODES, HALF), jnp.float32),
    )(z, w, deg2)


def _mid_body(t0_ref, t1_ref, deg_ref, w_ref, b_ref, o_ref):
    dinv = _dinv_of(deg_ref)
    t = jnp.concatenate([t0_ref[0], t1_ref[0]], axis=-1)
    a = t * dinv[:, None] + b_ref[...]
    r = jnp.maximum(a, 0.0)
    h = jnp.dot(r, w_ref[...], preferred_element_type=jnp.float32)
    o_ref[0] = h * dinv[:, None]


def _mid(t, deg2, w, b):
    blk = 1000
    return pl.pallas_call(
        _mid_body,
        grid=(N_NODES // blk, NC),
        in_specs=[
            pl.BlockSpec((1, blk, HALF), lambda i, j: (0, i, 0)),
            pl.BlockSpec((1, blk, HALF), lambda i, j: (1, i, 0)),
            pl.BlockSpec((blk, NC), lambda i, j: (i, 0)),
            pl.BlockSpec((EMBED_DIM, HALF), lambda i, j: (0, j)),
            pl.BlockSpec((1, EMBED_DIM), lambda i, j: (0, 0)),
        ],
        out_specs=pl.BlockSpec((1, blk, HALF), lambda i, j: (j, i, 0)),
        out_shape=jax.ShapeDtypeStruct((NC, N_NODES, HALF), jnp.float32),
    )(t, t, deg2, w, b)


def _final_body(t0_ref, t1_ref, deg_ref, b_ref, o_ref):
    dinv = _dinv_of(deg_ref)
    t = jnp.concatenate([t0_ref[0], t1_ref[0]], axis=-1)
    o_ref[...] = t * dinv[:, None] + b_ref[...]


def _final(t, deg2, b):
    blk = 1000
    return pl.pallas_call(
        _final_body,
        grid=(N_NODES // blk,),
        in_specs=[
            pl.BlockSpec((1, blk, HALF), lambda i: (0, i, 0)),
            pl.BlockSpec((1, blk, HALF), lambda i: (1, i, 0)),
            pl.BlockSpec((blk, NC), lambda i: (i, 0)),
            pl.BlockSpec((1, EMBED_DIM), lambda i: (0, 0)),
        ],
        out_specs=pl.BlockSpec((blk, EMBED_DIM), lambda i: (i, 0)),
        out_shape=jax.ShapeDtypeStruct((N_NODES, EMBED_DIM), jnp.float32),
    )(t, t, deg2, b)


def _loss_body(x_ref, y_ref, o_ref):
    i = pl.program_id(0)
    x = x_ref[...]
    y = y_ref[...]
    sx = jnp.sum(x * x, axis=1)
    sy = jnp.sum(y * y, axis=1)
    d = jnp.sum(x * y, axis=1)
    ix = 1.0 / jnp.maximum(jnp.sqrt(sx), 1e-12)
    iy = 1.0 / jnp.maximum(jnp.sqrt(sy), 1e-12)
    term = sx * ix * ix + sy * iy * iy - 2.0 * d * ix * iy
    p = (jnp.sum(term) * (1.0 / PAIRS)).reshape(1, 1)

    @pl.when(i == 0)
    def _():
        o_ref[...] = p

    @pl.when(i > 0)
    def _():
        o_ref[...] += p


def _loss(xy):
    blk = 512
    nblk = PAIRS // blk
    return pl.pallas_call(
        _loss_body,
        grid=(nblk,),
        in_specs=[
            pl.BlockSpec((blk, EMBED_DIM), lambda i: (i, 0)),
            pl.BlockSpec((blk, EMBED_DIM), lambda i: (i + nblk, 0)),
        ],
        out_specs=pl.BlockSpec((1, 1), lambda i: (0, 0)),
        out_shape=jax.ShapeDtypeStruct((1, 1), jnp.float32),
    )(xy, xy)


@functools.partial(
    pl.kernel,
    out_type=jax.ShapeDtypeStruct((NC * N_NODES,), jnp.float32),
    mesh=_sc_mesh,
    scratch_types=[
        pltpu.VMEM((CHUNK,), jnp.int32),
        pltpu.VMEM((DEG_TAIL,), jnp.int32),
        pltpu.VMEM((CHUNK,), jnp.float32),
        pltpu.VMEM((N_NODES,), jnp.float32),
        pltpu.VMEM_SHARED((N_NODES,), jnp.float32),
    ],
)
def _deg_kernel(dst_hbm, zeros_hbm, ones_hbm, deg_out,
                idx_v, tidx_v, ones_v, stage_v, acc_sh):
    c = lax.axis_index("c")
    s = lax.axis_index("s")
    base = (c * NS + s) * EPT_DEG

    @pl.when(s == 0)
    def _():
        pltpu.sync_copy(zeros_hbm, stage_v)
        pltpu.sync_copy(stage_v, acc_sh)

    pltpu.sync_copy(ones_hbm, ones_v)
    plsc.subcore_barrier()

    def body(k, carry):
        pltpu.sync_copy(dst_hbm.at[pl.ds(base + k * CHUNK, CHUNK)], idx_v)
        pltpu.sync_copy(ones_v, acc_sh.at[idx_v], add=True)
        return carry

    lax.fori_loop(0, DEG_FULL, body, 0)
    pltpu.sync_copy(dst_hbm.at[pl.ds(base + DEG_FULL * CHUNK, DEG_TAIL)], tidx_v)
    pltpu.sync_copy(ones_v.at[pl.ds(0, DEG_TAIL)], acc_sh.at[tidx_v], add=True)
    plsc.subcore_barrier()

    @pl.when(s == 0)
    def _():
        pltpu.sync_copy(acc_sh, stage_v)
        pltpu.sync_copy(stage_v, deg_out.at[pl.ds(c * N_NODES, N_NODES)])


@functools.partial(
    pl.kernel,
    out_type=jax.ShapeDtypeStruct((NC, N_NODES, HALF), jnp.float32),
    mesh=_sc_mesh,
    scratch_types=[
        pltpu.VMEM((CHUNK,), jnp.int32),
        pltpu.VMEM((CHUNK,), jnp.int32),
        pltpu.VMEM((AGG_TAIL,), jnp.int32),
        pltpu.VMEM((AGG_TAIL,), jnp.int32),
        pltpu.VMEM((CHUNK, HALF), jnp.float32),
        pltpu.VMEM((AGG_TAIL, HALF), jnp.float32),
        pltpu.VMEM_SHARED((N_NODES, HALF), jnp.float32),
    ],
)
def _agg_kernel(gflat_hbm, srcf_hbm, dst_hbm, t_out,
                sidx_v, didx_v, stidx_v, dtidx_v, rows_v, trows_v, acc_sh):
    c = lax.axis_index("c")
    s = lax.axis_index("s")
    rbase = s * ROWS_PER_TILE
    pltpu.sync_copy(
        gflat_hbm.at[pl.ds(c * N_NODES + rbase, ROWS_PER_TILE)],
        acc_sh.at[pl.ds(rbase, ROWS_PER_TILE)],
    )

    @pl.when(s == 0)
    def _():
        pltpu.sync_copy(
            gflat_hbm.at[pl.ds(c * N_NODES + NS * ROWS_PER_TILE, ROWS_REM)],
            acc_sh.at[pl.ds(NS * ROWS_PER_TILE, ROWS_REM)],
        )

    plsc.subcore_barrier()
    ebase = s * EPC

    def body(k, carry):
        pltpu.sync_copy(srcf_hbm.at[pl.ds(c * N_EDGES + ebase + k * CHUNK, CHUNK)], sidx_v)
        pltpu.sync_copy(dst_hbm.at[pl.ds(ebase + k * CHUNK, CHUNK)], didx_v)
        pltpu.sync_copy(gflat_hbm.at[sidx_v], rows_v)
        pltpu.sync_copy(rows_v, acc_sh.at[didx_v], add=True)
        return carry

    lax.fori_loop(0, AGG_FULL, body, 0)
    tail = AGG_FULL * CHUNK
    pltpu.sync_copy(srcf_hbm.at[pl.ds(c * N_EDGES + ebase + tail, AGG_TAIL)], stidx_v)
    pltpu.sync_copy(dst_hbm.at[pl.ds(ebase + tail, AGG_TAIL)], dtidx_v)
    pltpu.sync_copy(gflat_hbm.at[stidx_v], trows_v)
    pltpu.sync_copy(trows_v, acc_sh.at[dtidx_v], add=True)
    plsc.subcore_barrier()
    pltpu.sync_copy(
        acc_sh.at[pl.ds(rbase, ROWS_PER_TILE)],
        t_out.at[c, pl.ds(rbase, ROWS_PER_TILE)],
    )

    @pl.when(s == 0)
    def _():
        pltpu.sync_copy(
            acc_sh.at[pl.ds(NS * ROWS_PER_TILE, ROWS_REM)],
            t_out.at[c, pl.ds(NS * ROWS_PER_TILE, ROWS_REM)],
        )


@functools.partial(
    pl.kernel,
    out_type=jax.ShapeDtypeStruct((2 * PAIRS, EMBED_DIM), jnp.float32),
    mesh=_sc_mesh,
    scratch_types=[
        pltpu.VMEM((CHUNK,), jnp.int32),
        pltpu.VMEM((CHUNK, EMBED_DIM), jnp.float32),
    ],
)
def _pair_gather(emb_hbm, idx_hbm, out_hbm, idx_v, rows_v):
    c = lax.axis_index("c")
    s = lax.axis_index("s")
    w = c * NS + s

    def body(k, carry):
        b = w * GPT + k * CHUNK
        pltpu.sync_copy(idx_hbm.at[pl.ds(b, CHUNK)], idx_v)
        pltpu.sync_copy(emb_hbm.at[idx_v], rows_v)
        pltpu.sync_copy(rows_v, out_hbm.at[pl.ds(b, CHUNK)])
        return carry

    lax.fori_loop(0, GPT // CHUNK, body, 0)


def kernel(inputs, edge_index, item_emb, attr_emb,
           enc_W1, enc_b1, enc_W2, enc_b2,
           conv1_W, conv1_b, conv2_W, conv2_b):
    x = jnp.concatenate([item_emb, attr_emb], axis=0)
    src = edge_index[0]
    dst = edge_index[1]
    src2 = jnp.concatenate([src, src + N_NODES])
    idx_pairs = jnp.transpose(inputs).reshape(-1)
    zeros = jnp.zeros((N_NODES,), jnp.float32)
    ones = jnp.ones((CHUNK,), jnp.float32)

    z = _encoder(x, enc_W1, enc_b1.reshape(1, -1), enc_W2, enc_b2.reshape(1, -1))
    deg2 = _deg_kernel(dst, zeros, ones).reshape(NC, N_NODES).T
    g1 = _xform(z, conv1_W, deg2)
    t1 = _agg_kernel(g1.reshape(NC * N_NODES, HALF), src2, dst)
    g2 = _mid(t1, deg2, conv2_W, conv1_b.reshape(1, -1))
    t2 = _agg_kernel(g2.reshape(NC * N_NODES, HALF), src2, dst)
    emb = _final(t2, deg2, conv2_b.reshape(1, -1))
    xy = _pair_gather(emb, idx_pairs)
    loss = _loss(xy)[0, 0]
    return (loss, emb)
